# baseline (speedup 1.0000x reference)
import functools

import jax
import jax.numpy as jnp
from jax import lax
from jax.experimental import pallas as pl
from jax.experimental.pallas import tpu as pltpu
from jax.experimental.pallas import tpu_sc as plsc

_B = 16384
_NC = 2
_NS = 16
_NW = _NC * _NS
_BPW = _B // _NW
_L = 16
_NCHUNK = _BPW // _L
_NGROUP = 4
_GPC = _NCHUNK // _NGROUP

_LN2 = 0.6931471805599453
_SQRT2 = 1.4142135623730951


def _log_f32(x):
    bits = lax.bitcast_convert_type(x, jnp.int32)
    e = lax.shift_right_arithmetic(bits, 23) - 127
    mbits = jnp.bitwise_or(jnp.bitwise_and(bits, 0x007FFFFF), 0x3F800000)
    m = lax.bitcast_convert_type(mbits, jnp.float32)
    big = m > _SQRT2
    m = jnp.where(big, m * 0.5, m)
    e = jnp.where(big, e + 1, e)
    t = m - 1.0
    p = 0.114484355
    p = p * t - 0.18627697
    p = p * t + 0.20611785
    p = p * t - 0.2491121
    p = p * t + 0.33304814
    p = p * t - 0.50001293
    p = p * t + 1.0000031
    p = p * t + 3.342327e-08
    return e.astype(jnp.float32) * _LN2 + p


def _sc_body(vt_hbm, af_hbm, i0_hbm, i1_hbm, i2_hbm, i3_hbm, lp_hbm, snv_hbm,
             out_hbm, vt_v, af_v, i0_v, i1_v, i2_v, i3_v, lp_v, snv_v, out_v,
             sem, out_sem):
    wid = lax.axis_index("s") * _NC + lax.axis_index("c")
    base = wid * _BPW
    rows_sl = pl.ds(base, _BPW)
    copies = [
        pltpu.async_copy(vt_hbm.at[rows_sl], vt_v, sem),
        pltpu.async_copy(af_hbm.at[rows_sl], af_v, sem),
        pltpu.async_copy(i0_hbm.at[rows_sl], i0_v, sem),
        pltpu.async_copy(i1_hbm.at[rows_sl], i1_v, sem),
        pltpu.async_copy(i2_hbm.at[rows_sl], i2_v, sem),
        pltpu.async_copy(i3_hbm.at[rows_sl], i3_v, sem),
        pltpu.async_copy(lp_hbm, lp_v, sem),
        pltpu.async_copy(snv_hbm, snv_v, sem),
    ]
    for cp in copies:
        cp.wait()

    iota = lax.iota(jnp.int32, _L)

    def chunk(j):
        sl = pl.ds(j * _L, _L)
        rows = j * _L + iota
        vt = vt_v[sl]
        af = af_v[sl]
        flat = ((i0_v[sl] * 5 + i1_v[sl]) * 5 + i2_v[sl]) * 5 + i3_v[sl]
        snv = plsc.load_gather(snv_v, [flat])
        zero = jnp.zeros((_L,), jnp.float32)
        c0 = jnp.zeros((_L,), jnp.int32)
        lp0 = plsc.load_gather(lp_v, [vt, c0])
        lp1 = plsc.load_gather(lp_v, [vt, c0 + 1])
        lp4 = plsc.load_gather(lp_v, [vt, c0 + 4])
        lp0 = jnp.where(vt == 0, snv, lp0)
        lp3 = _log_f32(af * (2.0 - af))
        m = jnp.maximum(jnp.maximum(jnp.maximum(lp0, lp1), jnp.maximum(lp3, lp4)),
                        zero)
        s = (jnp.exp(lp0 - m) + jnp.exp(lp1 - m) + jnp.exp(zero - m)
             + jnp.exp(lp3 - m) + jnp.exp(lp4 - m))
        logz = m + _log_f32(s)
        plsc.store_scatter(out_v, [rows, c0], lp0 - logz)
        plsc.store_scatter(out_v, [rows, c0 + 1], lp1 - logz)
        plsc.store_scatter(out_v, [rows, c0 + 2], zero - logz)
        plsc.store_scatter(out_v, [rows, c0 + 3], lp3 - logz)
        plsc.store_scatter(out_v, [rows, c0 + 4], lp4 - logz)

    rpg = _GPC * _L
    out_copies = []
    for g in range(_NGROUP):
        plsc.parallel_loop(g * _GPC, (g + 1) * _GPC, unroll=1)(chunk)
        out_copies.append(pltpu.async_copy(
            out_v.at[pl.ds(g * rpg, rpg), :],
            out_hbm.at[pl.ds(base + g * rpg, rpg), :], out_sem))
    for cp in out_copies:
        cp.wait()


@functools.lru_cache(maxsize=None)
def _sc_call():
  return pl.kernel(
    _sc_body,
    out_type=jax.ShapeDtypeStruct((_B, 5), jnp.float32),
    mesh=plsc.VectorSubcoreMesh(core_axis_name="c", subcore_axis_name="s",
                                num_cores=_NC, num_subcores=_NS),
    compiler_params=pltpu.CompilerParams(needs_layout_passes=False),
    scratch_types=[
        pltpu.VMEM((_BPW,), jnp.int32),
        pltpu.VMEM((_BPW,), jnp.float32),
        pltpu.VMEM((_BPW,), jnp.int32),
        pltpu.VMEM((_BPW,), jnp.int32),
        pltpu.VMEM((_BPW,), jnp.int32),
        pltpu.VMEM((_BPW,), jnp.int32),
        pltpu.VMEM((5, 5), jnp.float32),
        pltpu.VMEM((625,), jnp.float32),
        pltpu.VMEM((_BPW, 5), jnp.float32),
        pltpu.SemaphoreType.DMA,
        pltpu.SemaphoreType.DMA,
    ],
  )


def kernel(variant_types_b, allele_frequencies_b, haplotypes_bs, log_priors_vc,
           somatic_snv_log_priors_rrra):
    seq_length = haplotypes_bs.shape[-1] // 2
    rc = (seq_length - 1) // 2
    return _sc_call()(
        variant_types_b.astype(jnp.int32),
        allele_frequencies_b,
        haplotypes_bs[:, rc - 1].astype(jnp.int32),
        haplotypes_bs[:, rc].astype(jnp.int32),
        haplotypes_bs[:, rc + 1].astype(jnp.int32),
        haplotypes_bs[:, rc + seq_length].astype(jnp.int32),
        log_priors_vc,
        somatic_snv_log_priors_rrra.reshape(625),
    )


# speedup vs baseline: 1.0036x; 1.0036x over previous
import functools

import jax
import jax.numpy as jnp
from jax import lax
from jax.experimental import pallas as pl
from jax.experimental.pallas import tpu as pltpu
from jax.experimental.pallas import tpu_sc as plsc

_B = 16384
_NC = 2
_NS = 16
_NW = _NC * _NS
_BPW = _B // _NW
_L = 16
_NCHUNK = _BPW // _L
_NGROUP = 2
_GPC = _NCHUNK // _NGROUP

_LN2 = 0.6931471805599453
_SQRT2 = 1.4142135623730951


def _log_f32(x):
    bits = lax.bitcast_convert_type(x, jnp.int32)
    e = lax.shift_right_arithmetic(bits, 23) - 127
    mbits = jnp.bitwise_or(jnp.bitwise_and(bits, 0x007FFFFF), 0x3F800000)
    m = lax.bitcast_convert_type(mbits, jnp.float32)
    big = m > _SQRT2
    m = jnp.where(big, m * 0.5, m)
    e = jnp.where(big, e + 1, e)
    t = m - 1.0
    p = 0.114484355
    p = p * t - 0.18627697
    p = p * t + 0.20611785
    p = p * t - 0.2491121
    p = p * t + 0.33304814
    p = p * t - 0.50001293
    p = p * t + 1.0000031
    p = p * t + 3.342327e-08
    return e.astype(jnp.float32) * _LN2 + p


def _sc_body(vt_hbm, af_hbm, i0_hbm, i1_hbm, i2_hbm, i3_hbm, lp_hbm, snv_hbm,
             out_hbm, vt_v, af_v, i0_v, i1_v, i2_v, i3_v, lp_v, snv_v, out_v,
             sem, out_sem):
    wid = lax.axis_index("s") * _NC + lax.axis_index("c")
    base = wid * _BPW
    rows_sl = pl.ds(base, _BPW)
    copies = [
        pltpu.async_copy(vt_hbm.at[rows_sl], vt_v, sem),
        pltpu.async_copy(af_hbm.at[rows_sl], af_v, sem),
        pltpu.async_copy(i0_hbm.at[rows_sl], i0_v, sem),
        pltpu.async_copy(i1_hbm.at[rows_sl], i1_v, sem),
        pltpu.async_copy(i2_hbm.at[rows_sl], i2_v, sem),
        pltpu.async_copy(i3_hbm.at[rows_sl], i3_v, sem),
        pltpu.async_copy(lp_hbm, lp_v, sem),
        pltpu.async_copy(snv_hbm, snv_v, sem),
    ]
    for cp in copies:
        cp.wait()

    iota = lax.iota(jnp.int32, _L)

    def chunk(j):
        sl = pl.ds(j * _L, _L)
        rows = j * _L + iota
        vt = vt_v[sl]
        af = af_v[sl]
        flat = ((i0_v[sl] * 5 + i1_v[sl]) * 5 + i2_v[sl]) * 5 + i3_v[sl]
        snv = plsc.load_gather(snv_v, [flat])
        zero = jnp.zeros((_L,), jnp.float32)
        c0 = jnp.zeros((_L,), jnp.int32)
        lp0 = plsc.load_gather(lp_v, [vt, c0])
        lp1 = plsc.load_gather(lp_v, [vt, c0 + 1])
        lp4 = plsc.load_gather(lp_v, [vt, c0 + 4])
        lp0 = jnp.where(vt == 0, snv, lp0)
        lp3 = _log_f32(af * (2.0 - af))
        m = jnp.maximum(jnp.maximum(jnp.maximum(lp0, lp1), jnp.maximum(lp3, lp4)),
                        zero)
        s = (jnp.exp(lp0 - m) + jnp.exp(lp1 - m) + jnp.exp(zero - m)
             + jnp.exp(lp3 - m) + jnp.exp(lp4 - m))
        logz = m + _log_f32(s)
        plsc.store_scatter(out_v, [rows, c0], lp0 - logz)
        plsc.store_scatter(out_v, [rows, c0 + 1], lp1 - logz)
        plsc.store_scatter(out_v, [rows, c0 + 2], zero - logz)
        plsc.store_scatter(out_v, [rows, c0 + 3], lp3 - logz)
        plsc.store_scatter(out_v, [rows, c0 + 4], lp4 - logz)

    rpg = _GPC * _L
    out_copies = []
    for g in range(_NGROUP):
        plsc.parallel_loop(g * _GPC, (g + 1) * _GPC, unroll=1)(chunk)
        out_copies.append(pltpu.async_copy(
            out_v.at[pl.ds(g * rpg, rpg), :],
            out_hbm.at[pl.ds(base + g * rpg, rpg), :], out_sem))
    for cp in out_copies:
        cp.wait()


@functools.lru_cache(maxsize=None)
def _sc_call():
  return pl.kernel(
    _sc_body,
    out_type=jax.ShapeDtypeStruct((_B, 5), jnp.float32),
    mesh=plsc.VectorSubcoreMesh(core_axis_name="c", subcore_axis_name="s",
                                num_cores=_NC, num_subcores=_NS),
    compiler_params=pltpu.CompilerParams(needs_layout_passes=False),
    scratch_types=[
        pltpu.VMEM((_BPW,), jnp.int32),
        pltpu.VMEM((_BPW,), jnp.float32),
        pltpu.VMEM((_BPW,), jnp.int32),
        pltpu.VMEM((_BPW,), jnp.int32),
        pltpu.VMEM((_BPW,), jnp.int32),
        pltpu.VMEM((_BPW,), jnp.int32),
        pltpu.VMEM((5, 5), jnp.float32),
        pltpu.VMEM((625,), jnp.float32),
        pltpu.VMEM((_BPW, 5), jnp.float32),
        pltpu.SemaphoreType.DMA,
        pltpu.SemaphoreType.DMA,
    ],
  )


def kernel(variant_types_b, allele_frequencies_b, haplotypes_bs, log_priors_vc,
           somatic_snv_log_priors_rrra):
    seq_length = haplotypes_bs.shape[-1] // 2
    rc = (seq_length - 1) // 2
    return _sc_call()(
        variant_types_b.astype(jnp.int32),
        allele_frequencies_b,
        haplotypes_bs[:, rc - 1].astype(jnp.int32),
        haplotypes_bs[:, rc].astype(jnp.int32),
        haplotypes_bs[:, rc + 1].astype(jnp.int32),
        haplotypes_bs[:, rc + seq_length].astype(jnp.int32),
        log_priors_vc,
        somatic_snv_log_priors_rrra.reshape(625),
    )
